# Initial kernel scaffold; baseline (speedup 1.0000x reference)
import jax
import jax.numpy as jnp
from jax.experimental import pallas as pl


def kernel(x, table, W, b):
    raise NotImplementedError("write your pallas kernel here")


# speedup vs baseline: 2.1461x; 2.1461x over previous
import functools

import numpy as np
import jax
import jax.numpy as jnp
from jax import lax
from jax.experimental import pallas as pl
from jax.experimental.pallas import tpu as pltpu
from jax.experimental.pallas import tpu_sc as plsc

V = 1000000
D = 64
OUT = 64
B = 16384
L = 50

NC = 2
NS = 16
NW = NC * NS
LANES = 16

ROWS_PER_W = B // NW
CB = 16
CHUNK_T = CB * L
N_CHUNKS = ROWS_PER_W // CB
IG = 100
NIG = CHUNK_T // IG
XROWS = B * L // IG

_GDN = lax.GatherDimensionNumbers(
    offset_dims=(), collapsed_slice_dims=(0,), start_index_map=(0,))


def _permute(v, idx):
    return lax.gather(v, idx.reshape(LANES, 1), _GDN, (1,),
                      mode=lax.GatherScatterMode.PROMISE_IN_BOUNDS)


def _xlane_sum(v, it):
    for p in (8, 4, 2, 1):
        v = v + _permute(v, it ^ p)
    return v


def _scales(s):
    i = lax.bitcast_convert_type(s, jnp.int32)
    i = 0x5F3759DF - lax.shift_right_logical(i, 1)
    y = lax.bitcast_convert_type(i, jnp.float32)
    sh = 0.5 * s
    for _ in range(3):
        y = y * (1.5 - sh * y * y)
    t = (s * y + 1e-7) * y
    r = y * (2.0 - t)
    return jnp.where(s > 1.0, r, 1.0)


def _make_sc_pool():
    mesh = plsc.VectorSubcoreMesh(core_axis_name="c", subcore_axis_name="s")

    @functools.partial(
        pl.kernel,
        mesh=mesh,
        out_type=jax.ShapeDtypeStruct((B, D), jnp.float32),
        scratch_types=[
            pltpu.VMEM((NIG, IG), jnp.int32),
            pltpu.VMEM((CHUNK_T, D), jnp.float32),
            pltpu.VMEM((CB, D), jnp.float32),
            pltpu.SemaphoreType.DMA,
        ],
        compiler_params=pltpu.CompilerParams(use_tc_tiling_on_sc=False),
    )
    def sc_pool(x_hbm, table_hbm, out_hbm, idx_v, rows_v, m_v, sem):
        wid = lax.axis_index("s") * NC + lax.axis_index("c")

        def chunk_body(ci, _):
            xrow = wid * (N_CHUNKS * NIG) + ci * NIG
            rbase = wid * ROWS_PER_W + ci * CB
            pltpu.sync_copy(x_hbm.at[pl.ds(xrow, NIG)], idx_v)
            cps = [
                pltpu.async_copy(
                    table_hbm.at[idx_v.at[k]],
                    rows_v.at[pl.ds(k * IG, IG)],
                    sem,
                )
                for k in range(NIG)
            ]
            for c in cps:
                c.wait()

            def row_body(rb, _):
                tb = rb * L
                it = lax.iota(jnp.int32, LANES)
                zi = it * 0
                zf = zi.astype(jnp.float32)
                one = zf + 1.0
                acc = [zf for _ in range(4)]
                off = 0
                for gsz in (8, 8, 8, 8, 8, 8, 2):
                    rows = []
                    s_pack = one
                    for i in range(gsz):
                        t = tb + off + i
                        v = [rows_v[t, pl.ds(j * LANES, LANES)]
                             for j in range(4)]
                        rows.append(v)
                        sq = v[0] * v[0] + v[1] * v[1]
                        sq = sq + v[2] * v[2]
                        sq = sq + v[3] * v[3]
                        ssp = _xlane_sum(sq, it)
                        s_pack = jnp.where(it == i, ssp, s_pack)
                    sc = _scales(s_pack)
                    for i in range(gsz):
                        si = _permute(sc, zi + i)
                        for j in range(4):
                            acc[j] = acc[j] + rows[i][j] * si
                    off += gsz
                inv = jnp.float32(1.0 / L)
                for j in range(4):
                    m_v[rb, pl.ds(j * LANES, LANES)] = acc[j] * inv
                return 0

            lax.fori_loop(0, CB, row_body, 0)
            pltpu.sync_copy(m_v, out_hbm.at[pl.ds(rbase, CB)])
            return 0

        lax.fori_loop(0, N_CHUNKS, chunk_body, 0)

    return sc_pool


_sc_pool = _make_sc_pool()


def _mm_body(m_ref, w_ref, b_ref, o_ref):
    o_ref[...] = (
        lax.dot_general(
            m_ref[...], w_ref[...], (((1,), (1,)), ((), ())),
            preferred_element_type=jnp.float32,
        )
        + b_ref[...]
    )


def _tc_linear(m, W, b):
    BM = 2048
    return pl.pallas_call(
        _mm_body,
        grid=(B // BM,),
        in_specs=[
            pl.BlockSpec((BM, D), lambda i: (i, 0)),
            pl.BlockSpec((OUT, D), lambda i: (0, 0)),
            pl.BlockSpec((1, OUT), lambda i: (0, 0)),
        ],
        out_specs=pl.BlockSpec((BM, OUT), lambda i: (i, 0)),
        out_shape=jax.ShapeDtypeStruct((B, OUT), jnp.float32),
    )(m, W, b)


@jax.jit
def kernel(x, table, W, b):
    x2d = x.astype(jnp.int32).reshape(XROWS, IG)
    m = _sc_pool(x2d, table)
    return _tc_linear(m, W, b.reshape(1, OUT))


# speedup vs baseline: 2.5605x; 1.1931x over previous
import functools

import jax
import jax.numpy as jnp
from jax import lax
from jax.experimental import pallas as pl
from jax.experimental.pallas import tpu as pltpu
from jax.experimental.pallas import tpu_sc as plsc

V = 1000000
D = 64
OUT = 64
B = 16384
L = 50

NC = 2
NS = 16
NW = NC * NS
LANES = 16

ROWS_PER_W = B // NW
CB = 16
CHUNK_T = CB * L
N_CHUNKS = ROWS_PER_W // CB

_GDN = lax.GatherDimensionNumbers(
    offset_dims=(), collapsed_slice_dims=(0,), start_index_map=(0,))


def _permute(v, idx):
    return lax.gather(v, idx.reshape(LANES, 1), _GDN, (1,),
                      mode=lax.GatherScatterMode.PROMISE_IN_BOUNDS)


def _xlane_sum(v, it):
    for p in (8, 4, 2, 1):
        v = v + _permute(v, it ^ p)
    return v


def _scales(s):
    i = lax.bitcast_convert_type(s, jnp.int32)
    i = 0x5F3759DF - lax.shift_right_logical(i, 1)
    y = lax.bitcast_convert_type(i, jnp.float32)
    sh = 0.5 * s
    for _ in range(3):
        y = y * (1.5 - sh * y * y)
    t = (s * y + 1e-7) * y
    r = y * (2.0 - t)
    return jnp.where(s > 1.0, r, 1.0)


def _make_sc_pool():
    mesh = plsc.VectorSubcoreMesh(core_axis_name="c", subcore_axis_name="s")

    @functools.partial(
        pl.kernel,
        mesh=mesh,
        out_type=jax.ShapeDtypeStruct((B, D), jnp.float32),
        scratch_types=[
            pltpu.VMEM((CB, L), jnp.int32),
            pltpu.VMEM((CB, L), jnp.int32),
            pltpu.VMEM((CHUNK_T, D), jnp.float32),
            pltpu.VMEM((CHUNK_T, D), jnp.float32),
            pltpu.VMEM((CB, D), jnp.float32),
            pltpu.SemaphoreType.DMA,
            pltpu.SemaphoreType.DMA,
        ],
        compiler_params=pltpu.CompilerParams(use_tc_tiling_on_sc=False),
    )
    def sc_pool(x_hbm, table_hbm, out_hbm, idx_a, idx_b, rows_a, rows_b,
                m_v, sem_a, sem_b):
        wid = lax.axis_index("s") * NC + lax.axis_index("c")
        row0 = wid * ROWS_PER_W

        def stage(ci, idx_v, rows_v, sem):
            pltpu.sync_copy(x_hbm.at[pl.ds(row0 + ci * CB, CB)], idx_v)
            for k in range(CB):
                pltpu.async_copy(
                    table_hbm.at[idx_v.at[k]],
                    rows_v.at[pl.ds(k * L, L)],
                    sem,
                )

        def drain(rows_v, sem):
            pltpu.make_async_copy(
                table_hbm.at[pl.ds(0, CHUNK_T)], rows_v, sem).wait()

        def compute(ci, rows_v):
            def row_body(rb, _):
                tb = rb * L
                it = lax.iota(jnp.int32, LANES)
                zi = it * 0
                zf = zi.astype(jnp.float32)
                acc = [zf for _ in range(4)]
                off = 0
                for gsz in (8, 8, 8, 8, 8, 8, 2):
                    rows = []
                    s_pack = zf + 1.0
                    for i in range(gsz):
                        t = tb + off + i
                        v = [rows_v[t, pl.ds(j * LANES, LANES)]
                             for j in range(4)]
                        rows.append(v)
                        sq = v[0] * v[0] + v[1] * v[1]
                        sq = sq + v[2] * v[2]
                        sq = sq + v[3] * v[3]
                        ssp = _xlane_sum(sq, it)
                        s_pack = jnp.where(it == i, ssp, s_pack)
                    sc = _scales(s_pack)
                    for i in range(gsz):
                        si = _permute(sc, zi + i)
                        for j in range(4):
                            acc[j] = acc[j] + rows[i][j] * si
                    off += gsz
                inv = jnp.float32(1.0 / L)
                for j in range(4):
                    m_v[rb, pl.ds(j * LANES, LANES)] = acc[j] * inv
                return 0

            lax.fori_loop(0, CB, row_body, 0)
            pltpu.sync_copy(m_v, out_hbm.at[pl.ds(row0 + ci * CB, CB)])

        stage(0, idx_a, rows_a, sem_a)

        def pair_body(p, _):
            ca = 2 * p
            stage(ca + 1, idx_b, rows_b, sem_b)
            drain(rows_a, sem_a)
            compute(ca, rows_a)

            @pl.when(ca + 2 < N_CHUNKS)
            def _():
                stage(ca + 2, idx_a, rows_a, sem_a)

            drain(rows_b, sem_b)
            compute(ca + 1, rows_b)
            return 0

        lax.fori_loop(0, N_CHUNKS // 2, pair_body, 0)

    return sc_pool


_sc_pool = _make_sc_pool()


def _mm_body(m_ref, w_ref, b_ref, o_ref):
    o_ref[...] = (
        lax.dot_general(
            m_ref[...], w_ref[...], (((1,), (1,)), ((), ())),
            preferred_element_type=jnp.float32,
        )
        + b_ref[...]
    )


def _tc_linear(m, W, b):
    BM = 2048
    return pl.pallas_call(
        _mm_body,
        grid=(B // BM,),
        in_specs=[
            pl.BlockSpec((BM, D), lambda i: (i, 0)),
            pl.BlockSpec((OUT, D), lambda i: (0, 0)),
            pl.BlockSpec((1, OUT), lambda i: (0, 0)),
        ],
        out_specs=pl.BlockSpec((BM, OUT), lambda i: (i, 0)),
        out_shape=jax.ShapeDtypeStruct((B, OUT), jnp.float32),
    )(m, W, b)


@jax.jit
def kernel(x, table, W, b):
    m = _sc_pool(x.astype(jnp.int32), table)
    return _tc_linear(m, W, b.reshape(1, OUT))
